# baseline (device time: 16616 ns/iter reference)
import jax
import jax.numpy as jnp
from jax import lax
from jax.experimental import pallas as pl
from jax.experimental.pallas import tpu as pltpu


def kernel(dy, W):
    m, k = dy.shape
    d = W.shape[0]

    def body(dy_ref, w_ref, out_ref, comm_ref, send_sem, recv_sem):
        my_x = lax.axis_index("x")
        my_y = lax.axis_index("y")
        my_z = lax.axis_index("z")
        nbr = (1 - my_x, my_y, my_z)

        barrier_sem = pltpu.get_barrier_semaphore()
        pl.semaphore_signal(
            barrier_sem, inc=1, device_id=nbr,
            device_id_type=pl.DeviceIdType.MESH,
        )
        pl.semaphore_wait(barrier_sem, 1)

        partial = lax.dot_general(
            dy_ref[:, :].astype(jnp.bfloat16),
            w_ref[:, :].astype(jnp.bfloat16),
            (((1,), (1,)), ((), ())),
            preferred_element_type=jnp.float32,
        )
        comm_ref[0, :, :] = partial.astype(jnp.bfloat16)

        rdma = pltpu.make_async_remote_copy(
            src_ref=comm_ref.at[0],
            dst_ref=comm_ref.at[1],
            send_sem=send_sem,
            recv_sem=recv_sem,
            device_id=nbr,
            device_id_type=pl.DeviceIdType.MESH,
        )
        rdma.start()
        rdma.wait()

        out_ref[:, :] = partial + comm_ref[1, :, :].astype(jnp.float32)

    return pl.pallas_call(
        body,
        out_shape=jax.ShapeDtypeStruct((m, d), jnp.float32),
        in_specs=[
            pl.BlockSpec(memory_space=pltpu.VMEM),
            pl.BlockSpec(memory_space=pltpu.VMEM),
        ],
        out_specs=pl.BlockSpec(memory_space=pltpu.VMEM),
        scratch_shapes=[
            pltpu.VMEM((2, m, d), jnp.bfloat16),
            pltpu.SemaphoreType.DMA,
            pltpu.SemaphoreType.DMA,
        ],
        compiler_params=pltpu.CompilerParams(collective_id=0),
    )(dy, W)


# device time: 16353 ns/iter; 1.0161x vs baseline; 1.0161x over previous
import jax
import jax.numpy as jnp
from jax import lax
from jax.experimental import pallas as pl
from jax.experimental.pallas import tpu as pltpu

C = 4


def kernel(dy, W):
    m, k = dy.shape
    d = W.shape[0]
    half = m // 2
    r = half // C

    def body(dy_ref, w_ref, out_ref,
             xsend_buf, xrecv_buf, ysend_buf, yrecv_buf,
             xsend_sems, xrecv_sems, ysend_sems, yrecv_sems):
        my_x = lax.axis_index("x")
        my_y = lax.axis_index("y")
        my_z = lax.axis_index("z")
        xnbr = (1 - my_x, my_y, my_z)
        ynbr = (my_x, 1 - my_y, my_z)

        barrier_sem = pltpu.get_barrier_semaphore()
        for nbr in (xnbr, ynbr):
            pl.semaphore_signal(
                barrier_sem, inc=1, device_id=nbr,
                device_id_type=pl.DeviceIdType.MESH,
            )
        pl.semaphore_wait(barrier_sem, 2)

        my_start = my_y * half
        p_my = lax.dot_general(
            dy_ref[pl.ds(my_start, half), :].astype(jnp.bfloat16),
            w_ref[:, :].astype(jnp.bfloat16),
            (((1,), (1,)), ((), ())),
            preferred_element_type=jnp.float32,
        )
        xsend_buf[:, :] = p_my.astype(jnp.bfloat16)

        xs = []
        for i in range(C):
            rdma = pltpu.make_async_remote_copy(
                src_ref=xsend_buf.at[pl.ds(i * r, r)],
                dst_ref=xrecv_buf.at[pl.ds(i * r, r)],
                send_sem=xsend_sems.at[i],
                recv_sem=xrecv_sems.at[i],
                device_id=xnbr,
                device_id_type=pl.DeviceIdType.MESH,
            )
            rdma.start()
            xs.append(rdma)

        ys = []
        for i in range(C):
            xs[i].wait_recv()
            sum_f32 = (p_my[i * r:(i + 1) * r, :]
                       + xrecv_buf[pl.ds(i * r, r)].astype(jnp.float32))
            out_ref[pl.ds(my_start + i * r, r), :] = sum_f32
            ysend_buf[pl.ds(i * r, r)] = sum_f32.astype(jnp.bfloat16)
            rdma = pltpu.make_async_remote_copy(
                src_ref=ysend_buf.at[pl.ds(i * r, r)],
                dst_ref=yrecv_buf.at[pl.ds(i * r, r)],
                send_sem=ysend_sems.at[i],
                recv_sem=yrecv_sems.at[i],
                device_id=ynbr,
                device_id_type=pl.DeviceIdType.MESH,
            )
            rdma.start()
            ys.append(rdma)

        other_start = (1 - my_y) * half
        for i in range(C):
            ys[i].wait_recv()
            out_ref[pl.ds(other_start + i * r, r), :] = (
                yrecv_buf[pl.ds(i * r, r)].astype(jnp.float32))

        for i in range(C):
            xs[i].wait_send()
            ys[i].wait_send()

    return pl.pallas_call(
        body,
        out_shape=jax.ShapeDtypeStruct((m, d), jnp.float32),
        in_specs=[
            pl.BlockSpec(memory_space=pltpu.VMEM),
            pl.BlockSpec(memory_space=pltpu.VMEM),
        ],
        out_specs=pl.BlockSpec(memory_space=pltpu.VMEM),
        scratch_shapes=[
            pltpu.VMEM((half, d), jnp.bfloat16),
            pltpu.VMEM((half, d), jnp.bfloat16),
            pltpu.VMEM((half, d), jnp.bfloat16),
            pltpu.VMEM((half, d), jnp.bfloat16),
            pltpu.SemaphoreType.DMA((C,)),
            pltpu.SemaphoreType.DMA((C,)),
            pltpu.SemaphoreType.DMA((C,)),
            pltpu.SemaphoreType.DMA((C,)),
        ],
        compiler_params=pltpu.CompilerParams(collective_id=0),
    )(dy, W)
